# Initial kernel scaffold; baseline (speedup 1.0000x reference)
import jax
import jax.numpy as jnp
from jax.experimental import pallas as pl


def kernel(h_cat, left_adj, right_adj, W_bilin, b_bilin, W_head, W_dep, W_r1, b_r1, W_r2, b_r2, roots):
    raise NotImplementedError("write your pallas kernel here")


# speedup vs baseline: 17.0254x; 17.0254x over previous
import jax
import jax.numpy as jnp
import numpy as np
from jax import lax
from jax.experimental import pallas as pl
from jax.experimental.pallas import tpu as pltpu

_ALPHA = 0.25
_Z = np.int32(0)


def _compat_kernel(h_ref, left_ref, right_ref, wb_ref, bb_ref, wh_ref, wd_ref,
                   wr1_ref, br1_ref, wr2_ref, br2_ref, r1h_ref,
                   lap_ref, gold_ref):
    f32 = jnp.float32
    h = h_ref[0]
    n = h.shape[0]

    dn_et = (((1,), (1,)), ((), ()))
    tmp0 = jnp.dot(h, wb_ref[0], preferred_element_type=f32)
    tmp1 = jnp.dot(h, wb_ref[1], preferred_element_type=f32)
    bilin0 = lax.dot_general(tmp0, h, dn_et, preferred_element_type=f32)
    bilin1 = lax.dot_general(tmp1, h, dn_et, preferred_element_type=f32)

    head0 = lax.dot_general(h, wh_ref[0:1, :], dn_et, preferred_element_type=f32)
    head1 = lax.dot_general(h, wh_ref[1:2, :], dn_et, preferred_element_type=f32)
    dep0 = lax.dot_general(wd_ref[0:1, :], h, dn_et, preferred_element_type=f32)
    dep1 = lax.dot_general(wd_ref[1:2, :], h, dn_et, preferred_element_type=f32)

    compat0 = bilin0 + head0 + dep0 + bb_ref[0, 0]
    compat1 = bilin1 + head1 + dep1 + bb_ref[0, 1]

    gold_c = (jnp.sum(compat0 * left_ref[0], axis=(0, 1), keepdims=True)
              + jnp.sum(compat1 * right_ref[0], axis=(0, 1), keepdims=True))

    a_mat = jnp.exp(compat0) + jnp.exp(compat1)

    z = jnp.dot(h, wr1_ref[...], preferred_element_type=f32) + br1_ref[0:1, :]
    z = 0.5 * z * (1.0 + lax.erf(z * f32(0.7071067811865476)))
    root_row = lax.dot_general(wr2_ref[...], z, (((0,), (1,)), ((), ())),
                               preferred_element_type=f32) + br2_ref[0, 0]
    gold_r = jnp.sum(root_row * r1h_ref[0], axis=(0, 1), keepdims=True)

    deg = jnp.sum(a_mat, axis=0, keepdims=True)
    ii = lax.broadcasted_iota(jnp.int32, (n, n), 0)
    jj = lax.broadcasted_iota(jnp.int32, (n, n), 1)
    lap = jnp.where(ii == jj, deg - a_mat, -a_mat)
    lap = jnp.where(ii == 0, jnp.exp(root_row), lap)
    lap_ref[0] = lap
    gold_ref[0] = jnp.broadcast_to(gold_c + gold_r, gold_ref.shape[1:])


def _lu_kernel(lap_ref, logabs_ref, sign_ref, mat_ref):
    f32 = jnp.float32
    b, n, _ = lap_ref.shape
    lap = lap_ref[...]
    s = jnp.max(jnp.abs(lap), axis=1, keepdims=True)
    mat_ref[...] = lap / s
    scale_log = jnp.sum(jnp.log(s), axis=2)

    iota_r = lax.broadcasted_iota(jnp.int32, (b, n, 1), 1)
    iota_l = lax.broadcasted_iota(jnp.int32, (b, n, n), 2)

    col0 = jnp.sum(jnp.where(iota_l == 0, mat_ref[...], f32(0.0)),
                   axis=2, keepdims=True)
    neg_inf = f32(-jnp.inf)

    def body(_, carry):
        k, col, logabs, sign = carry
        am = jnp.where(iota_r >= k, jnp.abs(col), neg_inf)
        m = jnp.max(am, axis=1, keepdims=True)
        piv = jnp.min(jnp.where(am == m, iota_r, jnp.int32(n)),
                      axis=1, keepdims=True)
        is_p = iota_r == piv
        is_k = iota_r == k
        pivot = jnp.sum(jnp.where(is_p, col, f32(0.0)), axis=1, keepdims=True)
        colk = jnp.sum(jnp.where(is_k, col, f32(0.0)), axis=1, keepdims=True)
        col_sw = jnp.where(is_p, colk, col)
        safe_pivot = jnp.where(pivot == 0.0, f32(1.0), pivot)
        factors = jnp.where(iota_r > k, col_sw, f32(0.0)) / safe_pivot

        matv = mat_ref[...]
        row_p = jnp.sum(jnp.where(is_p, matv, f32(0.0)), axis=1, keepdims=True)
        row_k = jnp.sum(jnp.where(is_k, matv, f32(0.0)), axis=1, keepdims=True)
        mat_sw = jnp.where(is_k, row_p, jnp.where(is_p, row_k, matv))
        new_mat = mat_sw - factors * row_p
        mat_ref[...] = new_mat
        col_next = jnp.sum(jnp.where(iota_l == k + 1, new_mat, f32(0.0)),
                           axis=2, keepdims=True)
        sign = sign * jnp.where(piv != k, f32(-1.0), f32(1.0))
        sign = sign * jnp.sign(pivot)
        logabs = logabs + jnp.log(jnp.abs(pivot))
        return jnp.int32(k + 1), col_next, logabs, sign

    init = (jnp.int32(0), col0,
            jnp.zeros((b, 1, 1), f32), jnp.ones((b, 1, 1), f32))
    _, _, logabs, sign = lax.fori_loop(0, n, body, init)
    logabs_ref[...] = jnp.broadcast_to(logabs[:, 0, :] + scale_log,
                                       logabs_ref.shape)
    sign_ref[...] = jnp.broadcast_to(sign[:, 0, :], sign_ref.shape)


def kernel(h_cat, left_adj, right_adj, W_bilin, b_bilin, W_head, W_dep,
           W_r1, b_r1, W_r2, b_r2, roots):
    f32 = jnp.float32
    b, n, h = h_cat.shape
    roots1h = jax.nn.one_hot(roots, n, dtype=f32)

    lap, gold = pl.pallas_call(
        _compat_kernel,
        grid=(b,),
        in_specs=[
            pl.BlockSpec((1, n, h), lambda i: (i, _Z, _Z)),
            pl.BlockSpec((1, n, n), lambda i: (i, _Z, _Z)),
            pl.BlockSpec((1, n, n), lambda i: (i, _Z, _Z)),
            pl.BlockSpec((2, h, h), lambda i: (_Z, _Z, _Z)),
            pl.BlockSpec((1, 2), lambda i: (_Z, _Z)),
            pl.BlockSpec((2, h), lambda i: (_Z, _Z)),
            pl.BlockSpec((2, h), lambda i: (_Z, _Z)),
            pl.BlockSpec((h, h), lambda i: (_Z, _Z)),
            pl.BlockSpec((1, h), lambda i: (_Z, _Z)),
            pl.BlockSpec((h, 1), lambda i: (_Z, _Z)),
            pl.BlockSpec((1, 1), lambda i: (_Z, _Z)),
            pl.BlockSpec((1, 1, n), lambda i: (i, _Z, _Z)),
        ],
        out_specs=[
            pl.BlockSpec((1, n, n), lambda i: (i, _Z, _Z)),
            pl.BlockSpec((1, 1, 128), lambda i: (i, _Z, _Z)),
        ],
        out_shape=[
            jax.ShapeDtypeStruct((b, n, n), f32),
            jax.ShapeDtypeStruct((b, 1, 128), f32),
        ],
    )(h_cat, left_adj, right_adj, W_bilin,
      b_bilin.reshape(1, 2).astype(f32), W_head, W_dep,
      W_r1, b_r1.reshape(1, h).astype(f32), W_r2,
      b_r2.reshape(1, 1).astype(f32), roots1h.reshape(b, 1, n))

    logabs, sign = pl.pallas_call(
        _lu_kernel,
        out_shape=[
            jax.ShapeDtypeStruct((b, 128), f32),
            jax.ShapeDtypeStruct((b, 128), f32),
        ],
        scratch_shapes=[pltpu.VMEM((b, n, n), f32)],
    )(lap)

    gold_v = gold[:, 0, 0].astype(jnp.float64)
    la = logabs[:, 0].astype(jnp.float64)
    sg = sign[:, 0]
    logdet = jnp.where(sg > 0, la, jnp.nan)
    valid = jnp.logical_and(~jnp.isnan(gold_v), ~jnp.isnan(logdet)).astype(jnp.float64)
    mask = (gold_v <= logdet * valid).astype(jnp.float64)
    loss = (logdet - gold_v) * mask
    loss = jnp.where(jnp.isnan(loss), 0.0, loss)
    return _ALPHA * jnp.sum(loss) / b


# speedup vs baseline: 26.7612x; 1.5718x over previous
import jax
import jax.numpy as jnp
import numpy as np
from jax import lax
from jax.experimental import pallas as pl
from jax.experimental.pallas import tpu as pltpu

_ALPHA = 0.25
_Z = np.int32(0)


def _compat_kernel(h_ref, left_ref, right_ref, wb_ref, bb_ref, wh_ref, wd_ref,
                   wr1_ref, br1_ref, wr2_ref, br2_ref, r1h_ref,
                   lap_ref, gold_ref):
    f32 = jnp.float32
    h = h_ref[0]
    n = h.shape[0]

    dn_et = (((1,), (1,)), ((), ()))
    tmp0 = jnp.dot(h, wb_ref[0], preferred_element_type=f32)
    tmp1 = jnp.dot(h, wb_ref[1], preferred_element_type=f32)
    bilin0 = lax.dot_general(tmp0, h, dn_et, preferred_element_type=f32)
    bilin1 = lax.dot_general(tmp1, h, dn_et, preferred_element_type=f32)

    head0 = lax.dot_general(h, wh_ref[0:1, :], dn_et, preferred_element_type=f32)
    head1 = lax.dot_general(h, wh_ref[1:2, :], dn_et, preferred_element_type=f32)
    dep0 = lax.dot_general(wd_ref[0:1, :], h, dn_et, preferred_element_type=f32)
    dep1 = lax.dot_general(wd_ref[1:2, :], h, dn_et, preferred_element_type=f32)

    compat0 = bilin0 + head0 + dep0 + bb_ref[0, 0]
    compat1 = bilin1 + head1 + dep1 + bb_ref[0, 1]

    gold_c = (jnp.sum(compat0 * left_ref[0], axis=(0, 1), keepdims=True)
              + jnp.sum(compat1 * right_ref[0], axis=(0, 1), keepdims=True))

    a_mat = jnp.exp(compat0) + jnp.exp(compat1)

    z = jnp.dot(h, wr1_ref[...], preferred_element_type=f32) + br1_ref[0:1, :]
    z = 0.5 * z * (1.0 + lax.erf(z * f32(0.7071067811865476)))
    root_row = lax.dot_general(wr2_ref[...], z, (((0,), (1,)), ((), ())),
                               preferred_element_type=f32) + br2_ref[0, 0]
    gold_r = jnp.sum(root_row * r1h_ref[0], axis=(0, 1), keepdims=True)

    deg = jnp.sum(a_mat, axis=0, keepdims=True)
    ii = lax.broadcasted_iota(jnp.int32, (n, n), 0)
    jj = lax.broadcasted_iota(jnp.int32, (n, n), 1)
    lap = jnp.where(ii == jj, deg - a_mat, -a_mat)
    lap = jnp.where(ii == 0, jnp.exp(root_row), lap)
    lap_ref[0] = lap
    gold_ref[0] = jnp.broadcast_to(gold_c + gold_r, gold_ref.shape[1:])


_CHUNK = 32


def _lu_kernel(lap_ref, logabs_ref, sign_ref, mat_ref):
    f32 = jnp.float32
    b, n, _ = lap_ref.shape
    lap = lap_ref[...]
    s = jnp.max(jnp.abs(lap), axis=1, keepdims=True)
    mat_ref[...] = lap / s
    scale_log = jnp.sum(jnp.log(s), axis=2)

    neg_inf = f32(-jnp.inf)
    logabs = jnp.zeros((b, 1, 1), f32)
    sign = jnp.ones((b, 1, 1), f32)

    for off in range(0, n, _CHUNK):
        nr = n - off
        iota_r = lax.broadcasted_iota(jnp.int32, (b, nr, 1), 1)
        iota_l = lax.broadcasted_iota(jnp.int32, (b, nr, nr), 2)
        col0 = jnp.sum(
            jnp.where(iota_l == 0, mat_ref[:, off:, off:], f32(0.0)),
            axis=2, keepdims=True)

        def body(_, carry, off=off, nr=nr, iota_r=iota_r, iota_l=iota_l):
            k, col, logabs, sign = carry
            am = jnp.where(iota_r >= k, jnp.abs(col), neg_inf)
            m = jnp.max(am, axis=1, keepdims=True)
            piv = jnp.min(jnp.where(am == m, iota_r, jnp.int32(nr)),
                          axis=1, keepdims=True)
            is_p = iota_r == piv
            is_k = iota_r == k
            pivot = jnp.sum(jnp.where(is_p, col, f32(0.0)), axis=1, keepdims=True)
            colk = jnp.sum(jnp.where(is_k, col, f32(0.0)), axis=1, keepdims=True)
            col_sw = jnp.where(is_p, colk, col)
            safe_pivot = jnp.where(pivot == 0.0, f32(1.0), pivot)
            factors = jnp.where(iota_r > k, col_sw, f32(0.0)) / safe_pivot

            matv = mat_ref[:, off:, off:]
            row_p = jnp.sum(jnp.where(is_p, matv, f32(0.0)), axis=1, keepdims=True)
            row_k = jnp.sum(jnp.where(is_k, matv, f32(0.0)), axis=1, keepdims=True)
            mat_sw = jnp.where(is_k, row_p, jnp.where(is_p, row_k, matv))
            new_mat = mat_sw - factors * row_p
            mat_ref[:, off:, off:] = new_mat
            col_next = jnp.sum(jnp.where(iota_l == k + 1, new_mat, f32(0.0)),
                               axis=2, keepdims=True)
            sign = sign * jnp.where(piv != k, f32(-1.0), f32(1.0))
            sign = sign * jnp.sign(pivot)
            logabs = logabs + jnp.log(jnp.abs(pivot))
            return jnp.int32(k + 1), col_next, logabs, sign

        init = (jnp.int32(0), col0, logabs, sign)
        _, _, logabs, sign = lax.fori_loop(0, _CHUNK, body, init)

    logabs_ref[...] = jnp.broadcast_to(logabs[:, 0, :] + scale_log,
                                       logabs_ref.shape)
    sign_ref[...] = jnp.broadcast_to(sign[:, 0, :], sign_ref.shape)


def kernel(h_cat, left_adj, right_adj, W_bilin, b_bilin, W_head, W_dep,
           W_r1, b_r1, W_r2, b_r2, roots):
    f32 = jnp.float32
    b, n, h = h_cat.shape
    roots1h = jax.nn.one_hot(roots, n, dtype=f32)

    lap, gold = pl.pallas_call(
        _compat_kernel,
        grid=(b,),
        in_specs=[
            pl.BlockSpec((1, n, h), lambda i: (i, _Z, _Z)),
            pl.BlockSpec((1, n, n), lambda i: (i, _Z, _Z)),
            pl.BlockSpec((1, n, n), lambda i: (i, _Z, _Z)),
            pl.BlockSpec((2, h, h), lambda i: (_Z, _Z, _Z)),
            pl.BlockSpec((1, 2), lambda i: (_Z, _Z)),
            pl.BlockSpec((2, h), lambda i: (_Z, _Z)),
            pl.BlockSpec((2, h), lambda i: (_Z, _Z)),
            pl.BlockSpec((h, h), lambda i: (_Z, _Z)),
            pl.BlockSpec((1, h), lambda i: (_Z, _Z)),
            pl.BlockSpec((h, 1), lambda i: (_Z, _Z)),
            pl.BlockSpec((1, 1), lambda i: (_Z, _Z)),
            pl.BlockSpec((1, 1, n), lambda i: (i, _Z, _Z)),
        ],
        out_specs=[
            pl.BlockSpec((1, n, n), lambda i: (i, _Z, _Z)),
            pl.BlockSpec((1, 1, 128), lambda i: (i, _Z, _Z)),
        ],
        out_shape=[
            jax.ShapeDtypeStruct((b, n, n), f32),
            jax.ShapeDtypeStruct((b, 1, 128), f32),
        ],
    )(h_cat, left_adj, right_adj, W_bilin,
      b_bilin.reshape(1, 2).astype(f32), W_head, W_dep,
      W_r1, b_r1.reshape(1, h).astype(f32), W_r2,
      b_r2.reshape(1, 1).astype(f32), roots1h.reshape(b, 1, n))

    logabs, sign = pl.pallas_call(
        _lu_kernel,
        out_shape=[
            jax.ShapeDtypeStruct((b, 128), f32),
            jax.ShapeDtypeStruct((b, 128), f32),
        ],
        scratch_shapes=[pltpu.VMEM((b, n, n), f32)],
    )(lap)

    gold_v = gold[:, 0, 0].astype(jnp.float64)
    la = logabs[:, 0].astype(jnp.float64)
    sg = sign[:, 0]
    logdet = jnp.where(sg > 0, la, jnp.nan)
    valid = jnp.logical_and(~jnp.isnan(gold_v), ~jnp.isnan(logdet)).astype(jnp.float64)
    mask = (gold_v <= logdet * valid).astype(jnp.float64)
    loss = (logdet - gold_v) * mask
    loss = jnp.where(jnp.isnan(loss), 0.0, loss)
    return _ALPHA * jnp.sum(loss) / b


# speedup vs baseline: 30.5700x; 1.1423x over previous
import jax
import jax.numpy as jnp
import numpy as np
from jax import lax
from jax.experimental import pallas as pl
from jax.experimental.pallas import tpu as pltpu

_ALPHA = 0.25
_Z = np.int32(0)


def _compat_kernel(h_ref, left_ref, right_ref, wb_ref, bb_ref, wh_ref, wd_ref,
                   wr1_ref, br1_ref, wr2_ref, br2_ref, r1h_ref,
                   lap_ref, gold_ref):
    f32 = jnp.float32
    h = h_ref[0]
    n = h.shape[0]

    dn_et = (((1,), (1,)), ((), ()))
    tmp0 = jnp.dot(h, wb_ref[0], preferred_element_type=f32)
    tmp1 = jnp.dot(h, wb_ref[1], preferred_element_type=f32)
    bilin0 = lax.dot_general(tmp0, h, dn_et, preferred_element_type=f32)
    bilin1 = lax.dot_general(tmp1, h, dn_et, preferred_element_type=f32)

    head0 = lax.dot_general(h, wh_ref[0:1, :], dn_et, preferred_element_type=f32)
    head1 = lax.dot_general(h, wh_ref[1:2, :], dn_et, preferred_element_type=f32)
    dep0 = lax.dot_general(wd_ref[0:1, :], h, dn_et, preferred_element_type=f32)
    dep1 = lax.dot_general(wd_ref[1:2, :], h, dn_et, preferred_element_type=f32)

    compat0 = bilin0 + head0 + dep0 + bb_ref[0, 0]
    compat1 = bilin1 + head1 + dep1 + bb_ref[0, 1]

    gold_c = (jnp.sum(compat0 * left_ref[0], axis=(0, 1), keepdims=True)
              + jnp.sum(compat1 * right_ref[0], axis=(0, 1), keepdims=True))

    a_mat = jnp.exp(compat0) + jnp.exp(compat1)

    z = jnp.dot(h, wr1_ref[...], preferred_element_type=f32) + br1_ref[0:1, :]
    z = 0.5 * z * (1.0 + lax.erf(z * f32(0.7071067811865476)))
    root_row = lax.dot_general(wr2_ref[...], z, (((0,), (1,)), ((), ())),
                               preferred_element_type=f32) + br2_ref[0, 0]
    gold_r = jnp.sum(root_row * r1h_ref[0], axis=(0, 1), keepdims=True)

    deg = jnp.sum(a_mat, axis=0, keepdims=True)
    ii = lax.broadcasted_iota(jnp.int32, (n, n), 0)
    jj = lax.broadcasted_iota(jnp.int32, (n, n), 1)
    lap = jnp.where(ii == jj, deg - a_mat, -a_mat)
    lap = jnp.where(ii == 0, jnp.exp(root_row), lap)
    lap_ref[0] = lap
    gold_ref[0] = jnp.broadcast_to(gold_c + gold_r, gold_ref.shape[1:])


_W = 32


def _bdot(lhs, rhs, lc, rc):
    return lax.dot_general(lhs, rhs, (((lc,), (rc,)), ((0,), (0,))),
                           preferred_element_type=jnp.float32,
                           precision=lax.Precision.HIGHEST)


def _lu_kernel(lap_ref, logabs_ref, sign_ref, mat_ref, pt_ref):
    f32 = jnp.float32
    b, n, _ = lap_ref.shape
    w = _W
    lap = lap_ref[...]
    s = jnp.max(jnp.abs(lap), axis=1, keepdims=True)
    mat_ref[...] = lap / s
    scale_log = jnp.sum(jnp.log(s), axis=2)

    neg_inf = f32(-jnp.inf)
    il = lax.broadcasted_iota(jnp.int32, (b, 1, n), 2)
    isub = lax.broadcasted_iota(jnp.int32, (b, n, 1), 1)
    js = lax.broadcasted_iota(jnp.int32, (b, w, 1), 1)
    eye_w = jnp.where(
        lax.broadcasted_iota(jnp.int32, (b, w, w), 1)
        == lax.broadcasted_iota(jnp.int32, (b, w, w), 2), f32(1.0), f32(0.0))

    logabs = jnp.zeros((b, 1, 1), f32)
    sign = jnp.ones((b, 1, 1), f32)
    rowsrc = lax.broadcasted_iota(jnp.int32, (b, n, 1), 1)

    for q in range(0, n, w):
        trail = n - q - w
        qmat = jnp.where(rowsrc == il, f32(1.0), f32(0.0))
        pt_ref[...] = _bdot(mat_ref[:, :, q:q + w], qmat, 1, 2)

        def body(_, carry, q=q):
            k, rowsrc, logabs, sign = carry
            g = q + k
            colk = pt_ref[:, pl.ds(k, 1), :]
            am = jnp.where(il >= g, jnp.abs(colk), neg_inf)
            m = jnp.max(am, axis=2, keepdims=True)
            piv = jnp.min(jnp.where(am == m, il, jnp.int32(n)),
                          axis=2, keepdims=True)
            is_p = il == piv
            is_g = il == g
            pivot = jnp.sum(jnp.where(is_p, colk, f32(0.0)), axis=2, keepdims=True)
            cg = jnp.sum(jnp.where(is_g, colk, f32(0.0)), axis=2, keepdims=True)
            colk_sw = jnp.where(is_p, cg, colk)
            safe_pivot = jnp.where(pivot == 0.0, f32(1.0), pivot)
            f = jnp.where(il > g, colk_sw, f32(0.0)) / safe_pivot

            ptv = pt_ref[...]
            sel_g = jnp.sum(jnp.where(is_g, ptv, f32(0.0)), axis=2, keepdims=True)
            sel_p = jnp.sum(jnp.where(is_p, ptv, f32(0.0)), axis=2, keepdims=True)
            pt_sw = jnp.where(is_g, sel_p, jnp.where(is_p, sel_g, ptv))
            upd = pt_sw - jnp.where(js > k, sel_p * f, f32(0.0))
            pt_ref[...] = jnp.where(js == k, f, upd)

            rs_g = jnp.sum(jnp.where(isub == g, rowsrc, 0), axis=1,
                           keepdims=True, dtype=jnp.int32)
            rs_p = jnp.sum(jnp.where(isub == piv, rowsrc, 0), axis=1,
                           keepdims=True, dtype=jnp.int32)
            rowsrc = jnp.where(isub == g, rs_p,
                               jnp.where(isub == piv, rs_g, rowsrc))

            sign = sign * jnp.where(piv != g, f32(-1.0), f32(1.0))
            sign = sign * jnp.sign(pivot)
            logabs = logabs + jnp.log(jnp.abs(pivot))
            return jnp.int32(k + 1), rowsrc, logabs, sign

        init = (jnp.int32(0), rowsrc, logabs, sign)
        _, rowsrc, logabs, sign = lax.fori_loop(0, w, body, init)

        if trail > 0:
            qmat2 = jnp.where(rowsrc == il, f32(1.0), f32(0.0))
            a12 = _bdot(qmat2[:, q:q + w, :], mat_ref[:, :, q + w:], 2, 1)
            nt = pt_ref[:, :, q:q + w]
            x = eye_w - nt
            at = eye_w + nt
            for _ in range(4):
                ax = _bdot(at, x, 2, 1)
                x = _bdot(x, 2.0 * eye_w - ax, 2, 1)
            u12 = _bdot(x, a12, 1, 1)
            l21t = _bdot(pt_ref[...], qmat2, 2, 1)
            schur = _bdot(l21t, u12, 1, 1)
            mat_ref[:, :, q + w:] = mat_ref[:, :, q + w:] - schur

    logabs_ref[...] = jnp.broadcast_to(logabs[:, 0, :] + scale_log,
                                       logabs_ref.shape)
    sign_ref[...] = jnp.broadcast_to(sign[:, 0, :], sign_ref.shape)


def kernel(h_cat, left_adj, right_adj, W_bilin, b_bilin, W_head, W_dep,
           W_r1, b_r1, W_r2, b_r2, roots):
    f32 = jnp.float32
    b, n, h = h_cat.shape
    roots1h = jax.nn.one_hot(roots, n, dtype=f32)

    lap, gold = pl.pallas_call(
        _compat_kernel,
        grid=(b,),
        in_specs=[
            pl.BlockSpec((1, n, h), lambda i: (i, _Z, _Z)),
            pl.BlockSpec((1, n, n), lambda i: (i, _Z, _Z)),
            pl.BlockSpec((1, n, n), lambda i: (i, _Z, _Z)),
            pl.BlockSpec((2, h, h), lambda i: (_Z, _Z, _Z)),
            pl.BlockSpec((1, 2), lambda i: (_Z, _Z)),
            pl.BlockSpec((2, h), lambda i: (_Z, _Z)),
            pl.BlockSpec((2, h), lambda i: (_Z, _Z)),
            pl.BlockSpec((h, h), lambda i: (_Z, _Z)),
            pl.BlockSpec((1, h), lambda i: (_Z, _Z)),
            pl.BlockSpec((h, 1), lambda i: (_Z, _Z)),
            pl.BlockSpec((1, 1), lambda i: (_Z, _Z)),
            pl.BlockSpec((1, 1, n), lambda i: (i, _Z, _Z)),
        ],
        out_specs=[
            pl.BlockSpec((1, n, n), lambda i: (i, _Z, _Z)),
            pl.BlockSpec((1, 1, 128), lambda i: (i, _Z, _Z)),
        ],
        out_shape=[
            jax.ShapeDtypeStruct((b, n, n), f32),
            jax.ShapeDtypeStruct((b, 1, 128), f32),
        ],
    )(h_cat, left_adj, right_adj, W_bilin,
      b_bilin.reshape(1, 2).astype(f32), W_head, W_dep,
      W_r1, b_r1.reshape(1, h).astype(f32), W_r2,
      b_r2.reshape(1, 1).astype(f32), roots1h.reshape(b, 1, n))

    logabs, sign = pl.pallas_call(
        _lu_kernel,
        out_shape=[
            jax.ShapeDtypeStruct((b, 128), f32),
            jax.ShapeDtypeStruct((b, 128), f32),
        ],
        scratch_shapes=[pltpu.VMEM((b, n, n), f32),
                        pltpu.VMEM((b, _W, n), f32)],
    )(lap)

    gold_v = gold[:, 0, 0].astype(jnp.float64)
    la = logabs[:, 0].astype(jnp.float64)
    sg = sign[:, 0]
    logdet = jnp.where(sg > 0, la, jnp.nan)
    valid = jnp.logical_and(~jnp.isnan(gold_v), ~jnp.isnan(logdet)).astype(jnp.float64)
    mask = (gold_v <= logdet * valid).astype(jnp.float64)
    loss = (logdet - gold_v) * mask
    loss = jnp.where(jnp.isnan(loss), 0.0, loss)
    return _ALPHA * jnp.sum(loss) / b


# speedup vs baseline: 51.6614x; 1.6899x over previous
import jax
import jax.numpy as jnp
import numpy as np
from jax import lax
from jax.experimental import pallas as pl
from jax.experimental.pallas import tpu as pltpu

_ALPHA = 0.25
_Z = np.int32(0)
_W = 32


def _compat_kernel(h_ref, leftT_ref, rightT_ref, wb_ref, bb_ref, wh_ref,
                   wd_ref, wr1_ref, br1_ref, wr2_ref, br2_ref, r1h_ref,
                   lapT_ref, gold_ref):
    f32 = jnp.float32
    h = h_ref[0]
    n = h.shape[0]

    dn_et = (((1,), (1,)), ((), ()))
    tmp0 = jnp.dot(h, wb_ref[0], preferred_element_type=f32)
    tmp1 = jnp.dot(h, wb_ref[1], preferred_element_type=f32)
    bilin0T = lax.dot_general(h, tmp0, dn_et, preferred_element_type=f32)
    bilin1T = lax.dot_general(h, tmp1, dn_et, preferred_element_type=f32)

    head0 = lax.dot_general(wh_ref[0:1, :], h, dn_et, preferred_element_type=f32)
    head1 = lax.dot_general(wh_ref[1:2, :], h, dn_et, preferred_element_type=f32)
    dep0 = lax.dot_general(h, wd_ref[0:1, :], dn_et, preferred_element_type=f32)
    dep1 = lax.dot_general(h, wd_ref[1:2, :], dn_et, preferred_element_type=f32)

    compat0T = bilin0T + head0 + dep0 + bb_ref[0, 0]
    compat1T = bilin1T + head1 + dep1 + bb_ref[0, 1]

    gold_c = (jnp.sum(compat0T * leftT_ref[0], axis=(0, 1), keepdims=True)
              + jnp.sum(compat1T * rightT_ref[0], axis=(0, 1), keepdims=True))

    aT = jnp.exp(compat0T) + jnp.exp(compat1T)

    z = jnp.dot(h, wr1_ref[...], preferred_element_type=f32) + br1_ref[0:1, :]
    z = 0.5 * z * (1.0 + lax.erf(z * f32(0.7071067811865476)))
    root_col = jnp.dot(z, wr2_ref[...], preferred_element_type=f32) + br2_ref[0, 0]
    gold_r = jnp.sum(root_col * r1h_ref[0], axis=(0, 1), keepdims=True)

    deg = jnp.sum(aT, axis=1, keepdims=True)
    jjd = lax.broadcasted_iota(jnp.int32, (n, n), 0)
    iid = lax.broadcasted_iota(jnp.int32, (n, n), 1)
    lapT = jnp.where(jjd == iid, deg - aT, -aT)
    lapT = jnp.where(iid == 0, jnp.exp(root_col), lapT)
    lapT_ref[0] = lapT
    gold_ref[0] = jnp.broadcast_to(gold_c + gold_r, gold_ref.shape[1:])


def _bdot(lhs, rhs, lc, rc):
    return lax.dot_general(lhs, rhs, (((lc,), (rc,)), ((0,), (0,))),
                           preferred_element_type=jnp.float32,
                           precision=lax.Precision.HIGHEST)


def _lu_kernel(lapT_ref, logabs_ref, sign_ref, mt_ref, qp_ref):
    f32 = jnp.float32
    b, n, _ = lapT_ref.shape
    w = _W
    lapT = lapT_ref[...]
    s = jnp.max(jnp.abs(lapT), axis=2, keepdims=True)
    mt_ref[...] = lapT / s
    scale_log = jnp.sum(jnp.log(s), axis=1)

    neg_inf = f32(-jnp.inf)
    il = lax.broadcasted_iota(jnp.int32, (b, 1, n), 2)
    isub = lax.broadcasted_iota(jnp.int32, (b, n, 1), 1)
    js = lax.broadcasted_iota(jnp.int32, (b, w, 1), 1)
    eye_w = jnp.where(
        lax.broadcasted_iota(jnp.int32, (b, w, w), 1)
        == lax.broadcasted_iota(jnp.int32, (b, w, w), 2), f32(1.0), f32(0.0))

    elim = jnp.zeros((b, 1, n), f32)
    signp = jnp.ones((b, 1, 1), f32)
    logabs = jnp.zeros((b, 1, 1), f32)
    pc = jnp.zeros((b, 1, n), f32)

    for q in range(0, n, w):
        trail = n - q - w

        def body(_, carry, q=q):
            k, elim, signp, logabs, pc = carry
            g = q + k
            colk = mt_ref[:, pl.ds(g, 1), :]
            am = jnp.where(elim > 0.5, neg_inf, jnp.abs(colk))
            m = jnp.max(am, axis=2, keepdims=True)
            piv = jnp.min(jnp.where(am == m, il, jnp.int32(n)),
                          axis=2, keepdims=True)
            is_p = il == piv
            pivot = jnp.sum(jnp.where(is_p, colk, f32(0.0)), axis=2, keepdims=True)
            safe_pivot = jnp.where(pivot == 0.0, f32(1.0), pivot)
            elim = elim + jnp.where(is_p, f32(1.0), f32(0.0))
            f = jnp.where(elim > 0.5, f32(0.0), colk) / safe_pivot

            ptv = mt_ref[:, q:q + w, :]
            u = jnp.sum(jnp.where(is_p, ptv, f32(0.0)), axis=2, keepdims=True)
            mt_ref[:, q:q + w, :] = jnp.where(
                js == k, f, ptv - jnp.where(js > k, u * f, f32(0.0)))
            qp_ref[:, pl.ds(k, 1), :] = jnp.where(is_p, f32(1.0), f32(0.0))

            pc = jnp.where(il == g, piv.astype(f32), pc)
            signp = signp * jnp.sign(pivot)
            logabs = logabs + jnp.log(jnp.abs(pivot))
            return jnp.int32(k + 1), elim, signp, logabs, pc

        init = (jnp.int32(0), elim, signp, logabs, pc)
        _, elim, signp, logabs, pc = lax.fori_loop(0, w, body, init)

        if trail > 0:
            fpan = mt_ref[:, q:q + w, :]
            qp = qp_ref[...]
            a12t = _bdot(mt_ref[:, q + w:, :], qp, 2, 2)
            nt = _bdot(fpan, qp, 2, 2)
            x = eye_w - nt
            at = eye_w + nt
            for _ in range(4):
                ax = _bdot(at, x, 2, 1)
                x = _bdot(x, 2.0 * eye_w - ax, 2, 1)
            u12t = _bdot(a12t, x, 2, 1)
            schur_t = _bdot(u12t, fpan, 2, 1)
            mt_ref[:, q + w:, :] = mt_ref[:, q + w:, :] - schur_t

    eye_n = jnp.where(isub == il, f32(1.0), f32(0.0))
    pr = _bdot(eye_n, pc, 2, 2)
    inv_cnt = jnp.sum(
        jnp.where(jnp.logical_and(isub < il, pr > pc), f32(1.0), f32(0.0)),
        axis=(1, 2), keepdims=True)
    inv_mod2 = inv_cnt - 2.0 * jnp.floor(inv_cnt * 0.5)
    parity = jnp.where(inv_mod2 > 0.5, f32(-1.0), f32(1.0))
    sign = signp * parity

    logabs_ref[...] = jnp.broadcast_to(logabs[:, 0, :] + scale_log,
                                       logabs_ref.shape)
    sign_ref[...] = jnp.broadcast_to(sign[:, 0, :], sign_ref.shape)


def kernel(h_cat, left_adj, right_adj, W_bilin, b_bilin, W_head, W_dep,
           W_r1, b_r1, W_r2, b_r2, roots):
    f32 = jnp.float32
    b, n, h = h_cat.shape
    roots1h = jax.nn.one_hot(roots, n, dtype=f32).reshape(b, n, 1)

    lapT, gold = pl.pallas_call(
        _compat_kernel,
        grid=(b,),
        in_specs=[
            pl.BlockSpec((1, n, h), lambda i: (i, _Z, _Z)),
            pl.BlockSpec((1, n, n), lambda i: (i, _Z, _Z)),
            pl.BlockSpec((1, n, n), lambda i: (i, _Z, _Z)),
            pl.BlockSpec((2, h, h), lambda i: (_Z, _Z, _Z)),
            pl.BlockSpec((1, 2), lambda i: (_Z, _Z)),
            pl.BlockSpec((2, h), lambda i: (_Z, _Z)),
            pl.BlockSpec((2, h), lambda i: (_Z, _Z)),
            pl.BlockSpec((h, h), lambda i: (_Z, _Z)),
            pl.BlockSpec((1, h), lambda i: (_Z, _Z)),
            pl.BlockSpec((h, 1), lambda i: (_Z, _Z)),
            pl.BlockSpec((1, 1), lambda i: (_Z, _Z)),
            pl.BlockSpec((1, n, 1), lambda i: (i, _Z, _Z)),
        ],
        out_specs=[
            pl.BlockSpec((1, n, n), lambda i: (i, _Z, _Z)),
            pl.BlockSpec((1, 1, 128), lambda i: (i, _Z, _Z)),
        ],
        out_shape=[
            jax.ShapeDtypeStruct((b, n, n), f32),
            jax.ShapeDtypeStruct((b, 1, 128), f32),
        ],
    )(h_cat, jnp.swapaxes(left_adj, 1, 2), jnp.swapaxes(right_adj, 1, 2),
      W_bilin, b_bilin.reshape(1, 2).astype(f32), W_head, W_dep,
      W_r1, b_r1.reshape(1, h).astype(f32), W_r2,
      b_r2.reshape(1, 1).astype(f32), roots1h)

    logabs, sign = pl.pallas_call(
        _lu_kernel,
        out_shape=[
            jax.ShapeDtypeStruct((b, 128), f32),
            jax.ShapeDtypeStruct((b, 128), f32),
        ],
        scratch_shapes=[pltpu.VMEM((b, n, n), f32),
                        pltpu.VMEM((b, _W, n), f32)],
    )(lapT)

    gold_v = gold[:, 0, 0].astype(jnp.float64)
    la = logabs[:, 0].astype(jnp.float64)
    sg = sign[:, 0]
    logdet = jnp.where(sg > 0, la, jnp.nan)
    valid = jnp.logical_and(~jnp.isnan(gold_v), ~jnp.isnan(logdet)).astype(jnp.float64)
    mask = (gold_v <= logdet * valid).astype(jnp.float64)
    loss = (logdet - gold_v) * mask
    loss = jnp.where(jnp.isnan(loss), 0.0, loss)
    return _ALPHA * jnp.sum(loss) / b
